# baseline (speedup 1.0000x reference)
import jax
import jax.numpy as jnp
from jax.experimental import pallas as pl
from jax.experimental.pallas import tpu as pltpu

CODEBOOK = 1024
DIM = 32
TBLK = 4096


def _vq_block(z_ref, emb_ref, zq_ref, ids_ref, acc_ref):
    zb = z_ref[0]
    emb = emb_ref[...]
    emb_sq = jnp.sum(emb * emb, axis=1, keepdims=True)
    z_sq = jnp.sum(zb * zb, axis=0, keepdims=True)
    m2scores = jax.lax.dot_general(
        emb * jnp.float32(-2.0), zb, (((1,), (0,)), ((), ())),
        preferred_element_type=jnp.float32)
    d = (z_sq + emb_sq) + m2scores
    dmin = jnp.min(d, axis=0)
    iota = jax.lax.broadcasted_iota(jnp.int32, d.shape, 0).astype(jnp.float32)
    sel = jnp.where(d == dmin[None, :], iota, jnp.float32(CODEBOOK))
    ids_f = jnp.min(sel, axis=0)
    ids_ref[0, 0, :] = ids_f.astype(jnp.int32)

    onehot = (iota == ids_f[None, :]).astype(jnp.float32)
    zq = jax.lax.dot_general(
        emb, onehot, (((0,), (0,)), ((), ())),
        preferred_element_type=jnp.float32)
    zq_ref[0] = zb + (zq - zb)

    part = jnp.sum(dmin).reshape(1, 1)
    step = pl.program_id(0) * pl.num_programs(1) + pl.program_id(1)

    @pl.when(step == 0)
    def _init():
        acc_ref[...] = part

    @pl.when(step != 0)
    def _accum():
        acc_ref[...] += part


@jax.jit
def _vq(z, embedding_table):
    B, C, H, W = z.shape
    T = H * W
    nblk = T // TBLK
    z3 = z.reshape(B, C, T)

    grid = (B, nblk)
    zq3, ids3, acc = pl.pallas_call(
        _vq_block,
        grid=grid,
        in_specs=[
            pl.BlockSpec((1, C, TBLK), lambda b, t: (b, 0, t)),
            pl.BlockSpec((CODEBOOK, DIM), lambda b, t: (0, 0)),
        ],
        out_specs=[
            pl.BlockSpec((1, C, TBLK), lambda b, t: (b, 0, t)),
            pl.BlockSpec((1, 1, TBLK), lambda b, t: (b * nblk + t, 0, 0)),
            pl.BlockSpec((1, 1), lambda b, t: (0, 0)),
        ],
        out_shape=[
            jax.ShapeDtypeStruct((B, C, T), jnp.float32),
            jax.ShapeDtypeStruct((B * nblk, 1, TBLK), jnp.int32),
            jax.ShapeDtypeStruct((1, 1), jnp.float32),
        ],
    )(z3, embedding_table)

    z_q = zq3.reshape(B, C, H, W)
    ids = ids3.reshape(B * T)
    mse = acc[0, 0] / (B * T * C)
    commitment_loss = 0.25 * mse
    codebook_loss = mse
    loss = commitment_loss + codebook_loss
    return (z_q, loss, commitment_loss, codebook_loss, ids)


def kernel(z, embedding_table):
    return _vq(z, embedding_table)


# speedup vs baseline: 1.0002x; 1.0002x over previous
import jax
import jax.numpy as jnp
from jax.experimental import pallas as pl

CODEBOOK = 1024
DIM = 32
TBLK = 4096


def _vq_block(z_ref, emb_ref, zq_ref, ids_ref, acc_ref):
    zb = z_ref[0]
    emb = emb_ref[...]
    emb_sq = jnp.sum(emb * emb, axis=1, keepdims=True)
    z_sq = jnp.sum(zb * zb, axis=0, keepdims=True)
    m2scores = jax.lax.dot_general(
        emb * jnp.float32(-2.0), zb, (((1,), (0,)), ((), ())),
        preferred_element_type=jnp.float32)
    d = (z_sq + emb_sq) + m2scores
    dmin = jnp.min(d, axis=0)
    iota = jax.lax.broadcasted_iota(jnp.int32, d.shape, 0).astype(jnp.float32)
    sel = jnp.where(d == dmin[None, :], iota, jnp.float32(CODEBOOK))
    ids_f = jnp.min(sel, axis=0)
    ids_ref[0, 0, :] = ids_f.astype(jnp.int32)

    onehot = (iota == ids_f[None, :]).astype(jnp.float32)
    zq = jax.lax.dot_general(
        emb, onehot, (((0,), (0,)), ((), ())),
        preferred_element_type=jnp.float32)
    zq_ref[0] = zb + (zq - zb)

    part = jnp.sum(dmin).reshape(1, 1)
    step = pl.program_id(0) * pl.num_programs(1) + pl.program_id(1)

    @pl.when(step == 0)
    def _init():
        acc_ref[...] = part

    @pl.when(step != 0)
    def _accum():
        acc_ref[...] += part


@jax.jit
def _vq(z, embedding_table):
    B, C, H, W = z.shape
    T = H * W
    nblk = T // TBLK
    z3 = z.reshape(B, C, T)

    grid = (B, nblk)
    zq3, ids3, acc = pl.pallas_call(
        _vq_block,
        grid=grid,
        in_specs=[
            pl.BlockSpec((1, C, TBLK), lambda b, t: (b, 0, t)),
            pl.BlockSpec((CODEBOOK, DIM), lambda b, t: (0, 0)),
        ],
        out_specs=[
            pl.BlockSpec((1, C, TBLK), lambda b, t: (b, 0, t)),
            pl.BlockSpec((1, 1, TBLK), lambda b, t: (b * nblk + t, 0, 0)),
            pl.BlockSpec((1, 1), lambda b, t: (0, 0)),
        ],
        out_shape=[
            jax.ShapeDtypeStruct((B, C, T), jnp.float32),
            jax.ShapeDtypeStruct((B * nblk, 1, TBLK), jnp.int32),
            jax.ShapeDtypeStruct((1, 1), jnp.float32),
        ],
    )(z3, embedding_table)

    z_q = zq3.reshape(B, C, H, W)
    ids = ids3.reshape(B * T)
    mse = acc[0, 0] / (B * T * C)
    commitment_loss = 0.25 * mse
    codebook_loss = mse
    loss = commitment_loss + codebook_loss
    return (z_q, loss, commitment_loss, codebook_loss, ids)


def kernel(z, embedding_table):
    return _vq(z, embedding_table)
